# baseline (speedup 1.0000x reference)
import jax
import jax.numpy as jnp
from jax.experimental import pallas as pl

EPS = 1e-5


def _mlp_kernel(x_ref, w0_ref, b0_ref, w1_ref, b1_ref, w2_ref, b2_ref,
                gamma_ref, beta_ref, out_ref):
    n = x_ref.shape[0]
    inv_n = jnp.float32(1.0 / n)
    gamma = gamma_ref[...]
    beta = beta_ref[...]

    h = jnp.dot(x_ref[...].astype(jnp.bfloat16), w0_ref[...],
                preferred_element_type=jnp.float32)
    h = h + b0_ref[...]
    mean = jnp.sum(h, axis=0, keepdims=True) * inv_n
    d = h - mean
    var = jnp.sum(d * d, axis=0, keepdims=True) * inv_n
    h = d * (gamma * jax.lax.rsqrt(var + EPS)) + beta
    h = jnp.tanh(h)

    h = jnp.dot(h.astype(jnp.bfloat16), w1_ref[...],
                preferred_element_type=jnp.float32)
    h = h + b1_ref[...]
    mean = jnp.sum(h, axis=0, keepdims=True) * inv_n
    d = h - mean
    var = jnp.sum(d * d, axis=0, keepdims=True) * inv_n
    h = d * (gamma * jax.lax.rsqrt(var + EPS)) + beta
    h = jnp.tanh(h)

    h = jnp.dot(h.astype(jnp.bfloat16), w2_ref[...],
                preferred_element_type=jnp.float32)
    out_ref[...] = h + b2_ref[...]


def kernel(x, edge_index, edge_attr, W0, b0, W1, b1, W2, b2, gamma, beta):
    del edge_index, edge_attr
    n, _ = x.shape
    d_out = W2.shape[0]
    return pl.pallas_call(
        _mlp_kernel,
        out_shape=jax.ShapeDtypeStruct((n, d_out), jnp.float32),
    )(
        x,
        W0.T.astype(jnp.bfloat16), b0[None, :],
        W1.T.astype(jnp.bfloat16), b1[None, :],
        W2.T.astype(jnp.bfloat16), b2[None, :],
        gamma[None, :], beta[None, :],
    )


# speedup vs baseline: 1.4155x; 1.4155x over previous
import jax
import jax.numpy as jnp
from jax.experimental import pallas as pl

EPS = 1e-5


def _bn_tanh(h, n, gamma, beta):
    inv_n = jnp.float32(1.0 / n)
    s = jnp.sum(h, axis=0, keepdims=True)
    q = jnp.sum(h * h, axis=0, keepdims=True)
    mean = s * inv_n
    var = q * inv_n - mean * mean
    scale = gamma * jax.lax.rsqrt(var + EPS)
    shift = beta - mean * scale
    return jnp.tanh(h * scale + shift)


def _mlp_kernel(x_ref, w0_ref, w1_ref, w2_ref, b2_ref, gamma_ref, beta_ref,
                out_ref):
    n = x_ref.shape[0]
    gamma = gamma_ref[...]
    beta = beta_ref[...]

    h = jnp.dot(x_ref[...], w0_ref[...], preferred_element_type=jnp.float32)
    h = _bn_tanh(h, n, gamma, beta)
    h = jnp.dot(h, w1_ref[...], preferred_element_type=jnp.float32)
    h = _bn_tanh(h, n, gamma, beta)
    h = jnp.dot(h, w2_ref[...], preferred_element_type=jnp.float32)
    out_ref[...] = h + b2_ref[...]


def kernel(x, edge_index, edge_attr, W0, b0, W1, b1, W2, b2, gamma, beta):
    del edge_index, edge_attr
    del b0, b1
    n, _ = x.shape
    d_out = W2.shape[0]
    return pl.pallas_call(
        _mlp_kernel,
        out_shape=jax.ShapeDtypeStruct((n, d_out), jnp.float32),
    )(
        x, W0.T, W1.T, W2.T, b2[None, :], gamma[None, :], beta[None, :],
    )
